# Initial kernel scaffold; baseline (speedup 1.0000x reference)
import jax
import jax.numpy as jnp
from jax.experimental import pallas as pl


def kernel(x, table, W1, b1, W2, b2):
    raise NotImplementedError("write your pallas kernel here")


# speedup vs baseline: 18.3285x; 18.3285x over previous
import jax
import jax.numpy as jnp
from jax import lax
from jax.experimental import pallas as pl
from jax.experimental.pallas import tpu as pltpu
from jax.experimental.pallas import tpu_sc as plsc

_NC = 2
_NS = 16
_NW = _NC * _NS

_B = 4096
_L = 200
_D = 100
_DP = 112
_NCH = _DP // 16
_SPW = _B // _NW
_FLUSH = 32
_C1 = 104
_C2 = 96

_MLP_BLK = 512


def _pool_body(x_hbm, table_hbm, out_hbm,
               idx_a, idx_b, buf_a, buf_b, out_v, sem):
    wid = lax.axis_index("c") * _NS + lax.axis_index("s")
    base = wid * _SPW

    def do_sample(s_local, f):
        s_global = base + s_local
        pltpu.sync_copy(x_hbm.at[s_global, pl.ds(0, _C1)], idx_a)
        pltpu.sync_copy(x_hbm.at[s_global, pl.ds(_C1, _C2)], idx_b)
        ca = pltpu.async_copy(table_hbm.at[idx_a], buf_a, sem)
        cb = pltpu.async_copy(table_hbm.at[idx_b], buf_b, sem)
        ca.wait()
        cb.wait()

        def make_group_body(buf):
            def group_body(g, accs):
                accs = list(accs)
                r0 = g * 4
                for rr in range(4):
                    for c in range(_NCH):
                        accs[c] = accs[c] + buf[r0 + rr, pl.ds(16 * c, 16)]
                return tuple(accs)
            return group_body

        zero = jnp.zeros((16,), jnp.float32)
        accs = lax.fori_loop(0, _C1 // 4, make_group_body(buf_a),
                             (zero,) * _NCH)
        accs = lax.fori_loop(0, _C2 // 4, make_group_body(buf_b), accs)
        for c in range(_NCH):
            out_v[f, pl.ds(16 * c, 16)] = accs[c]

    def flush_body(g, carry):
        def samp_body(f, carry2):
            do_sample(g * _FLUSH + f, f)
            return carry2
        lax.fori_loop(0, _FLUSH, samp_body, 0)
        pltpu.sync_copy(out_v, out_hbm.at[pl.ds(base + g * _FLUSH, _FLUSH)])
        return carry
    lax.fori_loop(0, _SPW // _FLUSH, flush_body, 0)


def _mlp_body(acc_ref, w1_ref, b1_ref, w2_ref, b2_ref, out_ref):
    a = acc_ref[...]
    h = jnp.maximum(
        lax.dot(a, w1_ref[...], preferred_element_type=jnp.float32)
        + b1_ref[...], 0.0)
    logits = lax.dot(h, w2_ref[...], preferred_element_type=jnp.float32) \
        + b2_ref[...]
    m = jnp.max(logits, axis=1, keepdims=True)
    e = jnp.exp(logits - m)
    out_ref[...] = e / jnp.sum(e, axis=1, keepdims=True)


def kernel(x, table, W1, b1, W2, b2):
    assert x.shape == (_B, _L) and table.shape[1] == _D
    hid = W1.shape[1]
    out_d = W2.shape[1]

    table_p = jnp.pad(table, ((0, 0), (0, _DP - _D)))

    mesh = plsc.VectorSubcoreMesh(
        core_axis_name="c", subcore_axis_name="s",
        num_cores=_NC, num_subcores=_NS)
    pool = pl.kernel(
        _pool_body,
        out_type=jax.ShapeDtypeStruct((_B, _DP), jnp.float32),
        mesh=mesh,
        scratch_types=[
            pltpu.VMEM((_C1,), jnp.int32),
            pltpu.VMEM((_C2,), jnp.int32),
            pltpu.VMEM((_C1, _DP), jnp.float32),
            pltpu.VMEM((_C2, _DP), jnp.float32),
            pltpu.VMEM((_FLUSH, _DP), jnp.float32),
            pltpu.SemaphoreType.DMA,
        ],
        compiler_params=pltpu.CompilerParams(
            use_tc_tiling_on_sc=False, needs_layout_passes=False),
    )
    acc = pool(x, table_p)

    w1p = jnp.concatenate(
        [W1, jnp.zeros((_DP - _D, hid), jnp.float32)], axis=0) * (1.0 / _L)
    probs = pl.pallas_call(
        _mlp_body,
        grid=(_B // _MLP_BLK,),
        in_specs=[
            pl.BlockSpec((_MLP_BLK, _DP), lambda i: (i, 0)),
            pl.BlockSpec((_DP, hid), lambda i: (0, 0)),
            pl.BlockSpec((1, hid), lambda i: (0, 0)),
            pl.BlockSpec((hid, out_d), lambda i: (0, 0)),
            pl.BlockSpec((1, out_d), lambda i: (0, 0)),
        ],
        out_specs=pl.BlockSpec((_MLP_BLK, out_d), lambda i: (i, 0)),
        out_shape=jax.ShapeDtypeStruct((_B, out_d), jnp.float32),
    )(acc, w1p, b1.reshape(1, hid), W2, b2.reshape(1, out_d))
    return probs


# speedup vs baseline: 26.8944x; 1.4674x over previous
import jax
import jax.numpy as jnp
from jax import lax
from jax.experimental import pallas as pl
from jax.experimental.pallas import tpu as pltpu
from jax.experimental.pallas import tpu_sc as plsc

_NC = 2
_NS = 16
_NW = _NC * _NS

_B = 4096
_L = 200
_D = 100
_DP = 112
_NCH = _DP // 16
_SPW = _B // _NW
_FLUSH = 32
_C1 = 104
_C2 = 96

_MLP_BLK = 512


def _pool_body(x_hbm, table_hbm, out_hbm,
               idx_v, buf0, buf1, out_v, sem0, sem1):
    wid = lax.axis_index("c") * _NS + lax.axis_index("s")
    base = wid * _SPW
    pltpu.sync_copy(x_hbm.at[pl.ds(base, _SPW)], idx_v)
    bufs = (buf0, buf1)
    sems = (sem0, sem1)

    def fire(s, b):
        pltpu.async_copy(
            table_hbm.at[idx_v.at[s, pl.ds(0, _C1)]],
            bufs[b].at[pl.ds(0, _C1)], sems[b])
        pltpu.async_copy(
            table_hbm.at[idx_v.at[s, pl.ds(_C1, _C2)]],
            bufs[b].at[pl.ds(_C1, _C2)], sems[b])

    def wait(b):
        pltpu.make_async_copy(
            table_hbm.at[pl.ds(0, _C1)],
            bufs[b].at[pl.ds(0, _C1)], sems[b]).wait()
        pltpu.make_async_copy(
            table_hbm.at[pl.ds(0, _C2)],
            bufs[b].at[pl.ds(_C1, _C2)], sems[b]).wait()

    def accumulate(buf, f):
        def group_body(g, accs):
            accs = list(accs)
            r0 = g * 4
            for rr in range(4):
                for c in range(_NCH):
                    accs[c] = accs[c] + buf[r0 + rr, pl.ds(16 * c, 16)]
            return tuple(accs)
        zero = jnp.zeros((16,), jnp.float32)
        accs = lax.fori_loop(0, _L // 4, group_body, (zero,) * _NCH)
        for c in range(_NCH):
            out_v[f, pl.ds(16 * c, 16)] = accs[c]

    fire(0, 0)

    def outer(i, carry):
        for b in range(2):
            s = 2 * i + b

            @pl.when(s + 1 < _SPW)
            def _():
                fire(s + 1, 1 - b)
            wait(b)
            accumulate(bufs[b], s)
        return carry
    lax.fori_loop(0, _SPW // 2, outer, 0)
    pltpu.sync_copy(out_v, out_hbm.at[pl.ds(base, _SPW)])


def _mlp_body(acc_ref, w1_ref, b1_ref, w2_ref, b2_ref, out_ref):
    a = acc_ref[...]
    h = jnp.maximum(
        lax.dot(a, w1_ref[...], preferred_element_type=jnp.float32)
        + b1_ref[...], 0.0)
    logits = lax.dot(h, w2_ref[...], preferred_element_type=jnp.float32) \
        + b2_ref[...]
    m = jnp.max(logits, axis=1, keepdims=True)
    e = jnp.exp(logits - m)
    out_ref[...] = e / jnp.sum(e, axis=1, keepdims=True)


def kernel(x, table, W1, b1, W2, b2):
    assert x.shape == (_B, _L) and table.shape[1] == _D
    hid = W1.shape[1]
    out_d = W2.shape[1]

    table_p = jnp.pad(table, ((0, 0), (0, _DP - _D)))

    mesh = plsc.VectorSubcoreMesh(
        core_axis_name="c", subcore_axis_name="s",
        num_cores=_NC, num_subcores=_NS)
    pool = pl.kernel(
        _pool_body,
        out_type=jax.ShapeDtypeStruct((_B, _DP), jnp.float32),
        mesh=mesh,
        scratch_types=[
            pltpu.VMEM((_SPW, _L), jnp.int32),
            pltpu.VMEM((_L, _DP), jnp.float32),
            pltpu.VMEM((_L, _DP), jnp.float32),
            pltpu.VMEM((_SPW, _DP), jnp.float32),
            pltpu.SemaphoreType.DMA,
            pltpu.SemaphoreType.DMA,
        ],
        compiler_params=pltpu.CompilerParams(
            use_tc_tiling_on_sc=False, needs_layout_passes=False),
    )
    acc = pool(x, table_p)

    w1p = jnp.concatenate(
        [W1, jnp.zeros((_DP - _D, hid), jnp.float32)], axis=0) * (1.0 / _L)
    probs = pl.pallas_call(
        _mlp_body,
        grid=(_B // _MLP_BLK,),
        in_specs=[
            pl.BlockSpec((_MLP_BLK, _DP), lambda i: (i, 0)),
            pl.BlockSpec((_DP, hid), lambda i: (0, 0)),
            pl.BlockSpec((1, hid), lambda i: (0, 0)),
            pl.BlockSpec((hid, out_d), lambda i: (0, 0)),
            pl.BlockSpec((1, out_d), lambda i: (0, 0)),
        ],
        out_specs=pl.BlockSpec((_MLP_BLK, out_d), lambda i: (i, 0)),
        out_shape=jax.ShapeDtypeStruct((_B, out_d), jnp.float32),
    )(acc, w1p, b1.reshape(1, hid), W2, b2.reshape(1, out_d))
    return probs


# speedup vs baseline: 32.8938x; 1.2231x over previous
import jax
import jax.numpy as jnp
from jax import lax
from jax.experimental import pallas as pl
from jax.experimental.pallas import tpu as pltpu
from jax.experimental.pallas import tpu_sc as plsc

_NC = 2
_NS = 16
_NW = _NC * _NS

_B = 4096
_L = 200
_D = 100
_DP = 112
_NCH = _DP // 16
_SPW = _B // _NW
_FLUSH = 32
_C1 = 104
_C2 = 96

_MLP_BLK = 512


def _pool_body(x_hbm, table_hbm, out_hbm,
               idx_v, buf0, buf1, out_v, sem0, sem1):
    wid = lax.axis_index("c") * _NS + lax.axis_index("s")
    base = wid * _SPW
    pltpu.sync_copy(x_hbm.at[pl.ds(base, _SPW)], idx_v)
    bufs = (buf0, buf1)
    sems = (sem0, sem1)

    def fire(s, b):
        pltpu.async_copy(
            table_hbm.at[idx_v.at[s, pl.ds(0, _C1)]],
            bufs[b].at[pl.ds(0, _C1)], sems[b])
        pltpu.async_copy(
            table_hbm.at[idx_v.at[s, pl.ds(_C1, _C2)]],
            bufs[b].at[pl.ds(_C1, _C2)], sems[b])

    def wait(b):
        pltpu.make_async_copy(
            table_hbm.at[pl.ds(0, _C1)],
            bufs[b].at[pl.ds(0, _C1)], sems[b]).wait()
        pltpu.make_async_copy(
            table_hbm.at[pl.ds(0, _C2)],
            bufs[b].at[pl.ds(_C1, _C2)], sems[b]).wait()

    def accumulate(buf, f):
        def group_body(g, accs):
            accs = list(accs)
            r0 = g * 4
            for rr in range(4):
                for c in range(_NCH):
                    accs[c] = accs[c] + buf[r0 + rr, pl.ds(16 * c, 16)]
            return tuple(accs)
        zero = jnp.zeros((16,), jnp.float32)
        accs = lax.fori_loop(0, _L // 4, group_body, (zero,) * _NCH)
        for c in range(_NCH):
            out_v[f, pl.ds(16 * c, 16)] = accs[c]

    fire(0, 0)

    def outer(i, carry):
        for b in range(2):
            s = 2 * i + b

            @pl.when(s + 1 < _SPW)
            def _():
                fire(s + 1, 1 - b)
            wait(b)
            accumulate(bufs[b], s)
        return carry
    lax.fori_loop(0, _SPW // 2, outer, 0)
    pltpu.sync_copy(out_v, out_hbm.at[pl.ds(base, _SPW)])


_PAD_BLK = 1024


def _pad_body(t_ref, o_ref):
    o_ref[:, :_D] = t_ref[...]
    o_ref[:, _D:] = jnp.zeros((_PAD_BLK, _DP - _D), jnp.float32)


def _mlp_body(acc_ref, w1_ref, b1_ref, w2_ref, b2_ref, out_ref):
    a = acc_ref[...]
    h = jnp.maximum(
        lax.dot(a, w1_ref[...], preferred_element_type=jnp.float32)
        + b1_ref[...], 0.0)
    logits = lax.dot(h, w2_ref[...], preferred_element_type=jnp.float32) \
        + b2_ref[...]
    m = jnp.max(logits, axis=1, keepdims=True)
    e = jnp.exp(logits - m)
    out_ref[...] = e / jnp.sum(e, axis=1, keepdims=True)


def kernel(x, table, W1, b1, W2, b2):
    assert x.shape == (_B, _L) and table.shape[1] == _D
    hid = W1.shape[1]
    out_d = W2.shape[1]

    vocab = table.shape[0]
    table_p = pl.pallas_call(
        _pad_body,
        grid=(pl.cdiv(vocab, _PAD_BLK),),
        in_specs=[pl.BlockSpec((_PAD_BLK, _D), lambda i: (i, 0))],
        out_specs=pl.BlockSpec((_PAD_BLK, _DP), lambda i: (i, 0)),
        out_shape=jax.ShapeDtypeStruct((vocab, _DP), jnp.float32),
    )(table)

    mesh = plsc.VectorSubcoreMesh(
        core_axis_name="c", subcore_axis_name="s",
        num_cores=_NC, num_subcores=_NS)
    pool = pl.kernel(
        _pool_body,
        out_type=jax.ShapeDtypeStruct((_B, _DP), jnp.float32),
        mesh=mesh,
        scratch_types=[
            pltpu.VMEM((_SPW, _L), jnp.int32),
            pltpu.VMEM((_L, _DP), jnp.float32),
            pltpu.VMEM((_L, _DP), jnp.float32),
            pltpu.VMEM((_SPW, _DP), jnp.float32),
            pltpu.SemaphoreType.DMA,
            pltpu.SemaphoreType.DMA,
        ],
        compiler_params=pltpu.CompilerParams(
            use_tc_tiling_on_sc=False, needs_layout_passes=False),
    )
    acc = pool(x, table_p)

    w1p = jnp.concatenate(
        [W1, jnp.zeros((_DP - _D, hid), jnp.float32)], axis=0) * (1.0 / _L)
    probs = pl.pallas_call(
        _mlp_body,
        grid=(_B // _MLP_BLK,),
        in_specs=[
            pl.BlockSpec((_MLP_BLK, _DP), lambda i: (i, 0)),
            pl.BlockSpec((_DP, hid), lambda i: (0, 0)),
            pl.BlockSpec((1, hid), lambda i: (0, 0)),
            pl.BlockSpec((hid, out_d), lambda i: (0, 0)),
            pl.BlockSpec((1, out_d), lambda i: (0, 0)),
        ],
        out_specs=pl.BlockSpec((_MLP_BLK, out_d), lambda i: (i, 0)),
        out_shape=jax.ShapeDtypeStruct((_B, out_d), jnp.float32),
    )(acc, w1p, b1.reshape(1, hid), W2, b2.reshape(1, out_d))
    return probs


# speedup vs baseline: 36.7263x; 1.1165x over previous
import jax
import jax.numpy as jnp
from jax import lax
from jax.experimental import pallas as pl
from jax.experimental.pallas import tpu as pltpu
from jax.experimental.pallas import tpu_sc as plsc

_NC = 2
_NS = 16
_NW = _NC * _NS

_B = 4096
_L = 200
_D = 100
_DP = 112
_NCH = _DP // 16
_SPW = _B // _NW
_FLUSH = 32
_C1 = 104
_C2 = 96

_MLP_BLK = 512


def _pool_body(x_hbm, table_hbm, out_hbm,
               idx_v, buf0, buf1, out_v, sem0, sem1):
    wid = lax.axis_index("c") * _NS + lax.axis_index("s")
    base = wid * _SPW
    pltpu.sync_copy(x_hbm.at[pl.ds(base, _SPW)], idx_v)
    bufs = (buf0, buf1)
    sems = (sem0, sem1)

    def fire(s, b):
        pltpu.async_copy(
            table_hbm.at[idx_v.at[s, pl.ds(0, _C1)]],
            bufs[b].at[pl.ds(0, _C1)], sems[b])
        pltpu.async_copy(
            table_hbm.at[idx_v.at[s, pl.ds(_C1, _C2)]],
            bufs[b].at[pl.ds(_C1, _C2)], sems[b])

    def wait(b):
        pltpu.make_async_copy(
            table_hbm.at[pl.ds(0, _C1)],
            bufs[b].at[pl.ds(0, _C1)], sems[b]).wait()
        pltpu.make_async_copy(
            table_hbm.at[pl.ds(0, _C2)],
            bufs[b].at[pl.ds(_C1, _C2)], sems[b]).wait()

    def accumulate(buf, f):
        def group_body(g, accs):
            accs = list(accs)
            r0 = g * 8
            for rr in range(8):
                for c in range(_NCH):
                    accs[c] = accs[c] + buf[r0 + rr, pl.ds(16 * c, 16)]
            return tuple(accs)
        zero = jnp.zeros((16,), jnp.float32)
        accs = lax.fori_loop(0, _L // 8, group_body, (zero,) * _NCH)
        for c in range(_NCH):
            out_v[f, pl.ds(16 * c, 16)] = accs[c]

    fire(0, 0)

    def outer(i, carry):
        for b in range(2):
            s = 2 * i + b

            @pl.when(s + 1 < _SPW)
            def _():
                fire(s + 1, 1 - b)
            wait(b)
            accumulate(bufs[b], s)
        return carry
    lax.fori_loop(0, _SPW // 2, outer, 0)
    pltpu.sync_copy(out_v, out_hbm.at[pl.ds(base, _SPW)])


_PAD_BLK = 4096


def _pad_body(t_ref, o_ref):
    o_ref[:, :_D] = t_ref[...]
    o_ref[:, _D:] = jnp.zeros((_PAD_BLK, _DP - _D), jnp.float32)


def _mlp_body(acc_ref, w1_ref, b1_ref, w2_ref, b2_ref, out_ref):
    a = acc_ref[...]
    h = jnp.maximum(
        lax.dot(a, w1_ref[...], preferred_element_type=jnp.float32)
        + b1_ref[...], 0.0)
    logits = lax.dot(h, w2_ref[...], preferred_element_type=jnp.float32) \
        + b2_ref[...]
    m = jnp.max(logits, axis=1, keepdims=True)
    e = jnp.exp(logits - m)
    out_ref[...] = e / jnp.sum(e, axis=1, keepdims=True)


def kernel(x, table, W1, b1, W2, b2):
    assert x.shape == (_B, _L) and table.shape[1] == _D
    hid = W1.shape[1]
    out_d = W2.shape[1]

    vocab = table.shape[0]
    table_p = pl.pallas_call(
        _pad_body,
        grid=(pl.cdiv(vocab, _PAD_BLK),),
        in_specs=[pl.BlockSpec((_PAD_BLK, _D), lambda i: (i, 0))],
        out_specs=pl.BlockSpec((_PAD_BLK, _DP), lambda i: (i, 0)),
        out_shape=jax.ShapeDtypeStruct((vocab, _DP), jnp.float32),
    )(table)

    mesh = plsc.VectorSubcoreMesh(
        core_axis_name="c", subcore_axis_name="s",
        num_cores=_NC, num_subcores=_NS)
    pool = pl.kernel(
        _pool_body,
        out_type=jax.ShapeDtypeStruct((_B, _DP), jnp.float32),
        mesh=mesh,
        scratch_types=[
            pltpu.VMEM((_SPW, _L), jnp.int32),
            pltpu.VMEM((_L, _DP), jnp.float32),
            pltpu.VMEM((_L, _DP), jnp.float32),
            pltpu.VMEM((_SPW, _DP), jnp.float32),
            pltpu.SemaphoreType.DMA,
            pltpu.SemaphoreType.DMA,
        ],
        compiler_params=pltpu.CompilerParams(
            use_tc_tiling_on_sc=False, needs_layout_passes=False),
    )
    acc = pool(x, table_p)

    w1p = jnp.concatenate(
        [W1, jnp.zeros((_DP - _D, hid), jnp.float32)], axis=0) * (1.0 / _L)
    probs = pl.pallas_call(
        _mlp_body,
        grid=(_B // _MLP_BLK,),
        in_specs=[
            pl.BlockSpec((_MLP_BLK, _DP), lambda i: (i, 0)),
            pl.BlockSpec((_DP, hid), lambda i: (0, 0)),
            pl.BlockSpec((1, hid), lambda i: (0, 0)),
            pl.BlockSpec((hid, out_d), lambda i: (0, 0)),
            pl.BlockSpec((1, out_d), lambda i: (0, 0)),
        ],
        out_specs=pl.BlockSpec((_MLP_BLK, out_d), lambda i: (i, 0)),
        out_shape=jax.ShapeDtypeStruct((_B, out_d), jnp.float32),
    )(acc, w1p, b1.reshape(1, hid), W2, b2.reshape(1, out_d))
    return probs


# speedup vs baseline: 38.0965x; 1.0373x over previous
import jax
import jax.numpy as jnp
from jax import lax
from jax.experimental import pallas as pl
from jax.experimental.pallas import tpu as pltpu
from jax.experimental.pallas import tpu_sc as plsc

_NC = 2
_NS = 16
_NW = _NC * _NS

_B = 4096
_L = 200
_D = 100
_DP = 112
_NCH = _DP // 16
_SPW = _B // _NW
_FLUSH = 32
_C1 = 104
_C2 = 96

_MLP_BLK = 512


def _pool_body(x_hbm, table_hbm, out_hbm,
               idx_v, buf0, buf1, out_v, semA0, semB0, semA1, semB1):
    wid = lax.axis_index("c") * _NS + lax.axis_index("s")
    base = wid * _SPW
    pltpu.sync_copy(x_hbm.at[pl.ds(base, _SPW)], idx_v)
    bufs = (buf0, buf1)
    semsA = (semA0, semA1)
    semsB = (semB0, semB1)

    def fire(s, b):
        pltpu.async_copy(
            table_hbm.at[idx_v.at[s, pl.ds(0, _C1)]],
            bufs[b].at[pl.ds(0, _C1)], semsA[b])
        pltpu.async_copy(
            table_hbm.at[idx_v.at[s, pl.ds(_C1, _C2)]],
            bufs[b].at[pl.ds(_C1, _C2)], semsB[b])

    def wait_a(b):
        pltpu.make_async_copy(
            table_hbm.at[pl.ds(0, _C1)],
            bufs[b].at[pl.ds(0, _C1)], semsA[b]).wait()

    def wait_b(b):
        pltpu.make_async_copy(
            table_hbm.at[pl.ds(0, _C2)],
            bufs[b].at[pl.ds(_C1, _C2)], semsB[b]).wait()

    def acc_range(buf, row0, ngroups, accs):
        def group_body(g, accs):
            accs = list(accs)
            r0 = row0 + g * 8
            for rr in range(8):
                for c in range(_NCH):
                    accs[c] = accs[c] + buf[r0 + rr, pl.ds(16 * c, 16)]
            return tuple(accs)
        return lax.fori_loop(0, ngroups, group_body, accs)

    fire(0, 0)

    def outer(i, carry):
        for b in range(2):
            s = 2 * i + b

            @pl.when(s + 1 < _SPW)
            def _():
                fire(s + 1, 1 - b)
            zero = jnp.zeros((16,), jnp.float32)
            wait_a(b)
            accs = acc_range(bufs[b], 0, _C1 // 8, (zero,) * _NCH)
            wait_b(b)
            accs = acc_range(bufs[b], _C1, _C2 // 8, accs)
            for c in range(_NCH):
                out_v[s, pl.ds(16 * c, 16)] = accs[c]
        return carry
    lax.fori_loop(0, _SPW // 2, outer, 0)
    pltpu.sync_copy(out_v, out_hbm.at[pl.ds(base, _SPW)])


_PAD_BLK = 4096


def _pad_body(t_ref, o_ref):
    o_ref[:, :_D] = t_ref[...]
    o_ref[:, _D:] = jnp.zeros((_PAD_BLK, _DP - _D), jnp.float32)


def _mlp_body(acc_ref, w1_ref, b1_ref, w2_ref, b2_ref, out_ref):
    a = acc_ref[...]
    h = jnp.maximum(
        lax.dot(a, w1_ref[...], preferred_element_type=jnp.float32)
        + b1_ref[...], 0.0)
    logits = lax.dot(h, w2_ref[...], preferred_element_type=jnp.float32) \
        + b2_ref[...]
    m = jnp.max(logits, axis=1, keepdims=True)
    e = jnp.exp(logits - m)
    out_ref[...] = e / jnp.sum(e, axis=1, keepdims=True)


def kernel(x, table, W1, b1, W2, b2):
    assert x.shape == (_B, _L) and table.shape[1] == _D
    hid = W1.shape[1]
    out_d = W2.shape[1]

    vocab = table.shape[0]
    table_p = pl.pallas_call(
        _pad_body,
        grid=(pl.cdiv(vocab, _PAD_BLK),),
        in_specs=[pl.BlockSpec((_PAD_BLK, _D), lambda i: (i, 0))],
        out_specs=pl.BlockSpec((_PAD_BLK, _DP), lambda i: (i, 0)),
        out_shape=jax.ShapeDtypeStruct((vocab, _DP), jnp.float32),
    )(table)

    mesh = plsc.VectorSubcoreMesh(
        core_axis_name="c", subcore_axis_name="s",
        num_cores=_NC, num_subcores=_NS)
    pool = pl.kernel(
        _pool_body,
        out_type=jax.ShapeDtypeStruct((_B, _DP), jnp.float32),
        mesh=mesh,
        scratch_types=[
            pltpu.VMEM((_SPW, _L), jnp.int32),
            pltpu.VMEM((_L, _DP), jnp.float32),
            pltpu.VMEM((_L, _DP), jnp.float32),
            pltpu.VMEM((_SPW, _DP), jnp.float32),
            pltpu.SemaphoreType.DMA,
            pltpu.SemaphoreType.DMA,
            pltpu.SemaphoreType.DMA,
            pltpu.SemaphoreType.DMA,
        ],
        compiler_params=pltpu.CompilerParams(
            use_tc_tiling_on_sc=False, needs_layout_passes=False),
    )
    acc = pool(x, table_p)

    w1p = jnp.concatenate(
        [W1, jnp.zeros((_DP - _D, hid), jnp.float32)], axis=0) * (1.0 / _L)
    probs = pl.pallas_call(
        _mlp_body,
        grid=(_B // _MLP_BLK,),
        in_specs=[
            pl.BlockSpec((_MLP_BLK, _DP), lambda i: (i, 0)),
            pl.BlockSpec((_DP, hid), lambda i: (0, 0)),
            pl.BlockSpec((1, hid), lambda i: (0, 0)),
            pl.BlockSpec((hid, out_d), lambda i: (0, 0)),
            pl.BlockSpec((1, out_d), lambda i: (0, 0)),
        ],
        out_specs=pl.BlockSpec((_MLP_BLK, out_d), lambda i: (i, 0)),
        out_shape=jax.ShapeDtypeStruct((_B, out_d), jnp.float32),
    )(acc, w1p, b1.reshape(1, hid), W2, b2.reshape(1, out_d))
    return probs
